# baseline (speedup 1.0000x reference)
import functools

import jax
import jax.numpy as jnp
from jax import lax
from jax.experimental import pallas as pl
from jax.experimental.pallas import tpu as pltpu
from jax.experimental.pallas import tpu_sc as plsc

N = 10000
E = 160000
EP = 163840
NPAD = 10240
POSP = 10016
NC, NS, L = 2, 16, 16
NW = NC * NS
EPT = EP // NW
NPT = NPAD // NW
CH = 2048
NCHUNK = EP // CH
RB = 128

@functools.cache
def _mesh():
    return plsc.VectorSubcoreMesh(core_axis_name="c", subcore_axis_name="s")


def _sc_a_body(posT, srcs, dsts, dstoff, d6, degp, posv, srcv, dstv, d6v,
               onesv, didx, zbuf, degsh):
    cid = lax.axis_index("c")
    sid = lax.axis_index("s")
    wid = sid * NC + cid
    base = wid * EPT
    rowbase = wid * (EPT // 128)

    def zero16(i, ref, dt):
        ref[pl.ds(i * 16, 16)] = jnp.zeros((16,), dt)

    lax.fori_loop(0, 1280 // 16, lambda i, _: (zero16(i, zbuf, jnp.float32), 0)[1], 0)
    for k in range(128 // 16):
        onesv[pl.ds(k * 16, 16)] = jnp.ones((16,), jnp.float32)

    if True:
        pltpu.sync_copy(zbuf, degsh.at[pl.ds(sid * 1280, 1280)])
        plsc.subcore_barrier()

        for s in range(2):
            pltpu.sync_copy(posT.at[s], posv)
            pltpu.sync_copy(srcs.at[s, pl.ds(base, EPT)], srcv)
            pltpu.sync_copy(dsts.at[s, pl.ds(base, EPT)], dstv)

            def d6_step(i, _):
                sv = srcv[pl.ds(i * 16, 16)]
                dv = dstv[pl.ds(i * 16, 16)]
                for c in range(3):
                    xs = plsc.load_gather(posv, [sv + c * POSP])
                    xd = plsc.load_gather(posv, [dv + c * POSP])
                    d6v[pl.ds(c * EPT + i * 16, 16)] = xs
                    d6v[pl.ds((3 + c) * EPT + i * 16, 16)] = xs - xd
                return 0

            lax.fori_loop(0, EPT // 16, d6_step, 0)
            for c in range(6):
                pltpu.sync_copy(d6v.at[pl.ds(c * EPT, EPT)],
                                d6.at[s, c, pl.ds(base, EPT)])

            def deg_step(j, _):
                pltpu.sync_copy(dstoff.at[s, pl.ds(rowbase + j, 1)], didx)
                pltpu.sync_copy(onesv, degsh.at[didx.at[0]], add=True)
                return 0

            lax.fori_loop(0, EPT // 128, deg_step, 0)

        plsc.subcore_barrier()
        pltpu.sync_copy(degsh.at[pl.ds(sid * 1280, 1280)],
                        degp.at[cid, pl.ds(sid * 1280, 1280)])


def _sc_a(posT, srcs, dsts, dstoff):
    return pl.kernel(
        _sc_a_body,
        out_type=(jax.ShapeDtypeStruct((2, 6, EP), jnp.float32),
                  jax.ShapeDtypeStruct((NC, 2 * NPAD), jnp.float32)),
        mesh=_mesh(),
        compiler_params=pltpu.CompilerParams(needs_layout_passes=False),
        scratch_types=[
            pltpu.VMEM((3 * POSP,), jnp.float32),
            pltpu.VMEM((EPT,), jnp.int32),
            pltpu.VMEM((EPT,), jnp.int32),
            pltpu.VMEM((6 * EPT,), jnp.float32),
            pltpu.VMEM((128,), jnp.float32),
            pltpu.VMEM((1, 128), jnp.int32),
            pltpu.VMEM((1280,), jnp.float32),
            pltpu.VMEM_SHARED((2 * NPAD,), jnp.float32),
        ],
    )(posT, srcs, dsts, dstoff)


def _sc_c_body(mflat, dstflat, hpad, acc, dstc, idl, dloc, rows, sem):
    cid = lax.axis_index("c")
    sid = lax.axis_index("s")
    wid = sid * NC + cid
    lo = wid * NPT
    hi = lo + NPT

    def zero_idl(i, _):
        idl[pl.ds(i * 16, 16)] = jnp.zeros((16,), jnp.int32)
        return 0

    lax.fori_loop(0, (CH + 16) // 16, zero_idl, 0)

    for s in range(2):
        def zero_acc(r, _):
            for g in range(16):
                acc[r, pl.ds(g * 16, 16)] = jnp.zeros((16,), jnp.float32)
            return 0

        lax.fori_loop(0, NPT, zero_acc, 0)

        def chunk_step(c, _):
            chunk_base = s * EP + c * CH
            pltpu.sync_copy(dstflat.at[pl.ds(chunk_base, CH)], dstc)

            def compact(i, cur):
                v = dstc[pl.ds(i * 16, 16)]
                msk = (v >= lo) & (v < hi)
                ids = lax.iota(jnp.int32, 16) + (chunk_base + i * 16)
                plsc.store_compressed(idl.at[pl.ds(cur, 16)], ids, mask=msk)
                plsc.store_compressed(dloc.at[pl.ds(cur, 16)], v - lo, mask=msk)
                cnt = jnp.max(plsc.all_reduce_population_count(msk))
                return cur + cnt

            cur = lax.fori_loop(0, CH // 16, compact, jnp.int32(0))
            nb = (cur + (RB - 1)) // RB

            def batch(j, _):
                pltpu.async_copy(mflat.at[idl.at[pl.ds(j * RB, RB)]], rows,
                                 sem).wait()
                rcnt = jnp.minimum(cur - j * RB, RB)

                def row_step(r, _):
                    d = dloc[pl.ds(j * RB + r, 16)][0]
                    for g in range(16):
                        a = acc[d, pl.ds(g * 16, 16)]
                        b = rows[r, pl.ds(g * 16, 16)]
                        acc[d, pl.ds(g * 16, 16)] = jnp.maximum(a, b)
                    return 0

                lax.fori_loop(0, rcnt, row_step, 0)
                return 0

            lax.fori_loop(0, nb, batch, 0)
            return 0

        lax.fori_loop(0, NCHUNK, chunk_step, 0)
        pltpu.sync_copy(acc, hpad.at[pl.ds(s * NPAD + lo, NPT)])


def _sc_c(mflat, dstflat):
    return pl.kernel(
        _sc_c_body,
        out_type=jax.ShapeDtypeStruct((2 * NPAD, 256), jnp.float32),
        mesh=_mesh(),
        compiler_params=pltpu.CompilerParams(needs_layout_passes=False),
        scratch_types=[
            pltpu.VMEM((NPT, 256), jnp.float32),
            pltpu.VMEM((CH,), jnp.int32),
            pltpu.VMEM((CH + 16,), jnp.int32),
            pltpu.VMEM((CH + 16,), jnp.int32),
            pltpu.VMEM((RB, 256), jnp.float32),
            pltpu.SemaphoreType.DMA,
        ],
    )(mflat, dstflat)


def _sc_e_body(yflat, srcr, dstr, S, rows, sidx, didx, zrows, sem, acc_sh):
    cid = lax.axis_index("c")
    sid = lax.axis_index("s")
    wid = sid * NC + cid
    rowbase = wid * (EPT // 128)

    def zero_zrows(r, _):
        for g in range(8):
            zrows[r, pl.ds(g * 16, 16)] = jnp.zeros((16,), jnp.float32)
        return 0

    lax.fori_loop(0, RB, zero_zrows, 0)

    if True:
        for s in range(2):
            for z in range(NPAD // NS // RB):
                pltpu.sync_copy(zrows, acc_sh.at[pl.ds(sid * 640 + z * RB, RB)])
            plsc.subcore_barrier()

            def gs_step(j, _):
                pltpu.sync_copy(srcr.at[s, pl.ds(rowbase + j, 1)], sidx)
                pltpu.sync_copy(dstr.at[s, pl.ds(rowbase + j, 1)], didx)
                pltpu.async_copy(yflat.at[sidx.at[0]], rows, sem).wait()
                pltpu.sync_copy(rows, acc_sh.at[didx.at[0]], add=True)
                return 0

            lax.fori_loop(0, EPT // 128, gs_step, 0)
            plsc.subcore_barrier()
            for z in range(NPAD // NS // RB):
                off = sid * 640 + z * RB
                pltpu.sync_copy(acc_sh.at[pl.ds(off, RB)],
                                S.at[cid, s, pl.ds(off, RB)])
            plsc.subcore_barrier()


def _sc_e(yflat, srcr, dstr):
    return pl.kernel(
        _sc_e_body,
        out_type=jax.ShapeDtypeStruct((NC, 2, NPAD, 128), jnp.float32),
        mesh=_mesh(),
        compiler_params=pltpu.CompilerParams(needs_layout_passes=False),
        scratch_types=[
            pltpu.VMEM((RB, 128), jnp.float32),
            pltpu.VMEM((1, 128), jnp.int32),
            pltpu.VMEM((1, 128), jnp.int32),
            pltpu.VMEM((RB, 128), jnp.float32),
            pltpu.SemaphoreType.DMA,
            pltpu.VMEM_SHARED((NPAD, 128), jnp.float32),
        ],
    )(yflat, srcr, dstr)


def _tc_b_body(d6_ref, w1_ref, b1_ref, w2_ref, b2_ref, m_ref):
    d6b = d6_ref[0]
    z = lax.dot_general(d6b, w1_ref[0], (((0,), (0,)), ((), ())),
                        preferred_element_type=jnp.float32)
    z = jax.nn.relu(z + b1_ref[0, 0])
    m = lax.dot_general(z, w2_ref[0], (((1,), (0,)), ((), ())),
                        preferred_element_type=jnp.float32)
    m_ref[0] = m + b2_ref[0, 0]


def _tc_b(d6, W1s, b1s, W2s, b2s):
    BE = 2048
    return pl.pallas_call(
        _tc_b_body,
        grid=(2, EP // BE),
        in_specs=[
            pl.BlockSpec((1, 6, BE), lambda s, e: (s, 0, e)),
            pl.BlockSpec((1, 6, 256), lambda s, e: (s, 0, 0)),
            pl.BlockSpec((1, 1, 256), lambda s, e: (s, 0, 0)),
            pl.BlockSpec((1, 256, 256), lambda s, e: (s, 0, 0)),
            pl.BlockSpec((1, 1, 256), lambda s, e: (s, 0, 0)),
        ],
        out_specs=pl.BlockSpec((1, BE, 256), lambda s, e: (s, e, 0)),
        out_shape=jax.ShapeDtypeStruct((2, EP, 256), jnp.float32),
    )(d6, W1s, b1s, W2s, b2s)


def _tc_d_body(hp0_ref, hp1_ref, dg0_ref, dg1_ref, wg_ref,
               y0_ref, y1_ref, dv0_ref, dv1_ref):
    h0 = hp0_ref[...]
    h1 = hp1_ref[...]
    wg = wg_ref[...]
    for s, (dg_ref, y_ref, dv_ref) in enumerate(
            ((dg0_ref, y0_ref, dv0_ref), (dg1_ref, y1_ref, dv1_ref))):
        deg = dg_ref[0] + dg_ref[1] + 1.0
        dinv = lax.rsqrt(deg)
        xw = (lax.dot_general(h0, wg[s, :256], (((1,), (0,)), ((), ())),
                              preferred_element_type=jnp.float32)
              + lax.dot_general(h1, wg[s, 256:], (((1,), (0,)), ((), ())),
                                preferred_element_type=jnp.float32))
        dvb = jnp.broadcast_to(dinv[:, None], xw.shape)
        y_ref[...] = dvb * xw
        dv_ref[...] = dvb


def _tc_d(hpad, degp, Wg):
    BN = 512
    G = NPAD // BN
    out = jax.ShapeDtypeStruct((NPAD, 128), jnp.float32)
    return pl.pallas_call(
        _tc_d_body,
        grid=(G,),
        in_specs=[
            pl.BlockSpec((BN, 256), lambda i: (i, 0)),
            pl.BlockSpec((BN, 256), lambda i: (i + G, 0)),
            pl.BlockSpec((2, BN), lambda i: (0, i)),
            pl.BlockSpec((2, BN), lambda i: (0, i + G)),
            pl.BlockSpec((2, 512, 128), lambda i: (0, 0, 0)),
        ],
        out_specs=[pl.BlockSpec((BN, 128), lambda i: (i, 0))] * 4,
        out_shape=[out, out, out, out],
    )(hpad, hpad, degp, degp, Wg)


def _tc_f_body(s0_ref, s1_ref, y0_ref, y1_ref, dv0_ref, dv1_ref,
               bg_ref, w3_ref, b3_ref, wct_ref, acc_ref):
    i = pl.program_id(0)
    g0 = jax.nn.relu(dv0_ref[...] * (s0_ref[0, 0] + s0_ref[1, 0]
                                     + y0_ref[...]) + bg_ref[0])
    g1 = jax.nn.relu(dv1_ref[...] * (s1_ref[0, 0] + s1_ref[1, 0]
                                     + y1_ref[...]) + bg_ref[1])
    h2 = jnp.concatenate([g0, g1], axis=1)
    h2 = jax.nn.relu(lax.dot_general(h2, w3_ref[...], (((1,), (0,)), ((), ())),
                                     preferred_element_type=jnp.float32)
                     + b3_ref[0])
    row = lax.broadcasted_iota(jnp.int32, (8, 128), 0)
    col = lax.broadcasted_iota(jnp.int32, (8, 128), 1)
    contrib = jnp.zeros((8, 128), jnp.float32)
    for c in range(3):
        val = jnp.sum(h2 * wct_ref[c])
        contrib = contrib + jnp.where((row == 0) & (col == c), val, 0.0)

    @pl.when(i == 0)
    def _():
        acc_ref[...] = contrib

    @pl.when(i > 0)
    def _():
        acc_ref[...] = acc_ref[...] + contrib


def _tc_f(S, y0, y1, dv0, dv1, bgs, W3, b3, WcT):
    BN = 512
    G = NPAD // BN
    return pl.pallas_call(
        _tc_f_body,
        grid=(G,),
        in_specs=[
            pl.BlockSpec((2, 1, BN, 128), lambda i: (0, 0, i, 0)),
            pl.BlockSpec((2, 1, BN, 128), lambda i: (0, 1, i, 0)),
            pl.BlockSpec((BN, 128), lambda i: (i, 0)),
            pl.BlockSpec((BN, 128), lambda i: (i, 0)),
            pl.BlockSpec((BN, 128), lambda i: (i, 0)),
            pl.BlockSpec((BN, 128), lambda i: (i, 0)),
            pl.BlockSpec((2, 128), lambda i: (0, 0)),
            pl.BlockSpec((256, 128), lambda i: (0, 0)),
            pl.BlockSpec((1, 128), lambda i: (0, 0)),
            pl.BlockSpec((3, BN, 128), lambda i: (0, i, 0)),
        ],
        out_specs=pl.BlockSpec((8, 128), lambda i: (0, 0)),
        out_shape=jax.ShapeDtypeStruct((8, 128), jnp.float32),
    )(S, S, y0, y1, dv0, dv1, bgs, W3, b3, WcT)


def kernel(pos_0, edge_index_0, batch_0, pos_1, edge_index_1, batch_1,
           W1_0, b1_0, W2_0, b2_0, W1_1, b1_1, W2_1, b2_1,
           Wg0, bg0, Wg1, bg1, W3, b3, Wc, bc):
    padN = EP - E
    srcs = jnp.stack([jnp.pad(edge_index_0[0], (0, padN)),
                      jnp.pad(edge_index_1[0], (0, padN))])
    dsts = jnp.stack([jnp.pad(edge_index_0[1], (0, padN), constant_values=N),
                      jnp.pad(edge_index_1[1], (0, padN), constant_values=N)])
    posT = jnp.stack([pos_0.T, pos_1.T])
    posT = jnp.pad(posT, ((0, 0), (0, 0), (0, POSP - N))).reshape(2, 3 * POSP)

    dstoff = (dsts + jnp.array([[0], [NPAD]], jnp.int32)).reshape(2, EP // 128, 128)
    dstflat = dsts.reshape(-1)
    srcr = (srcs + jnp.array([[0], [NPAD]], jnp.int32)).reshape(2, EP // 128, 128)
    dstr = dsts.reshape(2, EP // 128, 128)

    d6, degp = _sc_a(posT, srcs, dsts, dstoff)

    W1s = jnp.stack([W1_0, W1_1])
    b1s = jnp.stack([b1_0, b1_1]).reshape(2, 1, 256)
    W2s = jnp.stack([W2_0, W2_1])
    b2s = jnp.stack([b2_0, b2_1]).reshape(2, 1, 256)
    m = _tc_b(d6, W1s, b1s, W2s, b2s)
    mflat = m.reshape(2 * EP, 256)

    hpad = _sc_c(mflat, dstflat)

    Wg = jnp.stack([Wg0, Wg1])
    y0, y1, dv0, dv1 = _tc_d(hpad, degp, Wg)
    yflat = jnp.concatenate([y0, y1], axis=0)

    S = _sc_e(yflat, srcr, dstr)

    bgs = jnp.stack([bg0, bg1])
    WcT = (jnp.pad(Wc, ((0, (NPAD - N) * 128), (0, 0)))
           .reshape(NPAD, 128, 3).transpose(2, 0, 1))
    acc = _tc_f(S, y0, y1, dv0, dv1, bgs, W3, b3.reshape(1, 128), WcT)
    return acc[0, :3] + bc


# speedup vs baseline: 1.0269x; 1.0269x over previous
import functools

import jax
import jax.numpy as jnp
from jax import lax
from jax.experimental import pallas as pl
from jax.experimental.pallas import tpu as pltpu
from jax.experimental.pallas import tpu_sc as plsc

N = 10000
E = 160000
EP = 163840
NPAD = 10240
POSP = 10016
NC, NS, L = 2, 16, 16
NW = NC * NS
EPT = EP // NW
NPT = NPAD // NW
CH = 2048
NCHUNK = EP // CH
RB = 128

@functools.cache
def _mesh():
    return plsc.VectorSubcoreMesh(core_axis_name="c", subcore_axis_name="s")


def _sc_a_body(posT, srcs, dsts, dstoff, d6, degp, posv, srcv, dstv, d6v,
               onesv, didx, zbuf, degsh):
    cid = lax.axis_index("c")
    sid = lax.axis_index("s")
    wid = sid * NC + cid
    base = wid * EPT
    rowbase = wid * (EPT // 128)

    def zero16(i, ref, dt):
        ref[pl.ds(i * 16, 16)] = jnp.zeros((16,), dt)

    lax.fori_loop(0, 1280 // 16, lambda i, _: (zero16(i, zbuf, jnp.float32), 0)[1], 0)
    for k in range(128 // 16):
        onesv[pl.ds(k * 16, 16)] = jnp.ones((16,), jnp.float32)

    if True:
        pltpu.sync_copy(zbuf, degsh.at[pl.ds(sid * 1280, 1280)])
        plsc.subcore_barrier()

        for s in range(2):
            pltpu.sync_copy(posT.at[s], posv)
            pltpu.sync_copy(srcs.at[s, pl.ds(base, EPT)], srcv)
            pltpu.sync_copy(dsts.at[s, pl.ds(base, EPT)], dstv)

            def d6_step(i, _):
                sv = srcv[pl.ds(i * 16, 16)]
                dv = dstv[pl.ds(i * 16, 16)]
                for c in range(3):
                    xs = plsc.load_gather(posv, [sv + c * POSP])
                    xd = plsc.load_gather(posv, [dv + c * POSP])
                    d6v[pl.ds(c * EPT + i * 16, 16)] = xs
                    d6v[pl.ds((3 + c) * EPT + i * 16, 16)] = xs - xd
                return 0

            lax.fori_loop(0, EPT // 16, d6_step, 0)
            for c in range(6):
                pltpu.sync_copy(d6v.at[pl.ds(c * EPT, EPT)],
                                d6.at[s, c, pl.ds(base, EPT)])

            def deg_step(j, _):
                pltpu.sync_copy(dstoff.at[s, pl.ds(rowbase + j, 1)], didx)
                pltpu.sync_copy(onesv, degsh.at[didx.at[0]], add=True)
                return 0

            lax.fori_loop(0, EPT // 128, deg_step, 0)

        plsc.subcore_barrier()
        pltpu.sync_copy(degsh.at[pl.ds(sid * 1280, 1280)],
                        degp.at[cid, pl.ds(sid * 1280, 1280)])


def _sc_a(posT, srcs, dsts, dstoff):
    return pl.kernel(
        _sc_a_body,
        out_type=(jax.ShapeDtypeStruct((2, 6, EP), jnp.float32),
                  jax.ShapeDtypeStruct((NC, 2 * NPAD), jnp.float32)),
        mesh=_mesh(),
        compiler_params=pltpu.CompilerParams(needs_layout_passes=False),
        scratch_types=[
            pltpu.VMEM((3 * POSP,), jnp.float32),
            pltpu.VMEM((EPT,), jnp.int32),
            pltpu.VMEM((EPT,), jnp.int32),
            pltpu.VMEM((6 * EPT,), jnp.float32),
            pltpu.VMEM((128,), jnp.float32),
            pltpu.VMEM((1, 128), jnp.int32),
            pltpu.VMEM((1280,), jnp.float32),
            pltpu.VMEM_SHARED((2 * NPAD,), jnp.float32),
        ],
    )(posT, srcs, dsts, dstoff)


def _sc_c_body(mflat, dstflat, hpad, acc, dstc, idl, dloc, rows, sem):
    cid = lax.axis_index("c")
    sid = lax.axis_index("s")
    wid = sid * NC + cid
    lo = wid * NPT
    hi = lo + NPT

    def zero_idl(i, _):
        idl[pl.ds(i * 16, 16)] = jnp.zeros((16,), jnp.int32)
        return 0

    lax.fori_loop(0, (CH + 16) // 16, zero_idl, 0)

    for s in range(2):
        def zero_acc(r, _):
            for g in range(16):
                acc[r, pl.ds(g * 16, 16)] = jnp.zeros((16,), jnp.float32)
            return 0

        lax.fori_loop(0, NPT, zero_acc, 0)

        def chunk_step(c, _):
            chunk_base = s * EP + c * CH
            pltpu.sync_copy(dstflat.at[pl.ds(chunk_base, CH)], dstc)

            def compact(i, cur):
                v = dstc[pl.ds(i * 16, 16)]
                msk = (v >= lo) & (v < hi)
                ids = lax.iota(jnp.int32, 16) + (chunk_base + i * 16)
                plsc.store_compressed(idl.at[pl.ds(cur, 16)], ids, mask=msk)
                plsc.store_compressed(dloc.at[pl.ds(cur, 16)], v - lo, mask=msk)
                cnt = plsc.all_reduce_population_count(msk)[0]
                return cur + cnt

            cur = lax.fori_loop(0, CH // 16, compact, jnp.int32(0))
            nb = (cur + (RB - 1)) // RB

            def batch(j, _):
                pltpu.async_copy(mflat.at[idl.at[pl.ds(j * RB, RB)]], rows,
                                 sem).wait()
                rcnt = jnp.minimum(cur - j * RB, RB)

                def row_step(r, _):
                    d = dloc[pl.ds(j * RB + r, 16)][0]
                    avs = [acc[d, pl.ds(g * 16, 16)] for g in range(16)]
                    bvs = [rows[r, pl.ds(g * 16, 16)] for g in range(16)]
                    for g in range(16):
                        acc[d, pl.ds(g * 16, 16)] = jnp.maximum(avs[g], bvs[g])
                    return 0

                lax.fori_loop(0, rcnt, row_step, 0)
                return 0

            lax.fori_loop(0, nb, batch, 0)
            return 0

        lax.fori_loop(0, NCHUNK, chunk_step, 0)
        pltpu.sync_copy(acc, hpad.at[pl.ds(s * NPAD + lo, NPT)])


def _sc_c(mflat, dstflat):
    return pl.kernel(
        _sc_c_body,
        out_type=jax.ShapeDtypeStruct((2 * NPAD, 256), jnp.float32),
        mesh=_mesh(),
        compiler_params=pltpu.CompilerParams(needs_layout_passes=False),
        scratch_types=[
            pltpu.VMEM((NPT, 256), jnp.float32),
            pltpu.VMEM((CH,), jnp.int32),
            pltpu.VMEM((CH + 16,), jnp.int32),
            pltpu.VMEM((CH + 16,), jnp.int32),
            pltpu.VMEM((RB, 256), jnp.float32),
            pltpu.SemaphoreType.DMA,
        ],
    )(mflat, dstflat)


def _sc_e_body(yflat, srcr, dstr, S, rows, sidx, didx, zrows, sem, acc_sh):
    cid = lax.axis_index("c")
    sid = lax.axis_index("s")
    wid = sid * NC + cid
    rowbase = wid * (EPT // 128)

    def zero_zrows(r, _):
        for g in range(8):
            zrows[r, pl.ds(g * 16, 16)] = jnp.zeros((16,), jnp.float32)
        return 0

    lax.fori_loop(0, RB, zero_zrows, 0)

    if True:
        for s in range(2):
            for z in range(NPAD // NS // RB):
                pltpu.sync_copy(zrows, acc_sh.at[pl.ds(sid * 640 + z * RB, RB)])
            plsc.subcore_barrier()

            def gs_step(j, _):
                pltpu.sync_copy(srcr.at[s, pl.ds(rowbase + j, 1)], sidx)
                pltpu.sync_copy(dstr.at[s, pl.ds(rowbase + j, 1)], didx)
                pltpu.async_copy(yflat.at[sidx.at[0]], rows, sem).wait()
                pltpu.sync_copy(rows, acc_sh.at[didx.at[0]], add=True)
                return 0

            lax.fori_loop(0, EPT // 128, gs_step, 0)
            plsc.subcore_barrier()
            for z in range(NPAD // NS // RB):
                off = sid * 640 + z * RB
                pltpu.sync_copy(acc_sh.at[pl.ds(off, RB)],
                                S.at[cid, s, pl.ds(off, RB)])
            plsc.subcore_barrier()


def _sc_e(yflat, srcr, dstr):
    return pl.kernel(
        _sc_e_body,
        out_type=jax.ShapeDtypeStruct((NC, 2, NPAD, 128), jnp.float32),
        mesh=_mesh(),
        compiler_params=pltpu.CompilerParams(needs_layout_passes=False),
        scratch_types=[
            pltpu.VMEM((RB, 128), jnp.float32),
            pltpu.VMEM((1, 128), jnp.int32),
            pltpu.VMEM((1, 128), jnp.int32),
            pltpu.VMEM((RB, 128), jnp.float32),
            pltpu.SemaphoreType.DMA,
            pltpu.VMEM_SHARED((NPAD, 128), jnp.float32),
        ],
    )(yflat, srcr, dstr)


def _tc_b_body(d6_ref, w1_ref, b1_ref, w2_ref, b2_ref, m_ref):
    d6b = d6_ref[0]
    z = lax.dot_general(d6b, w1_ref[0], (((0,), (0,)), ((), ())),
                        preferred_element_type=jnp.float32)
    z = jax.nn.relu(z + b1_ref[0, 0])
    m = lax.dot_general(z, w2_ref[0], (((1,), (0,)), ((), ())),
                        preferred_element_type=jnp.float32)
    m_ref[0] = m + b2_ref[0, 0]


def _tc_b(d6, W1s, b1s, W2s, b2s):
    BE = 2048
    return pl.pallas_call(
        _tc_b_body,
        grid=(2, EP // BE),
        in_specs=[
            pl.BlockSpec((1, 6, BE), lambda s, e: (s, 0, e)),
            pl.BlockSpec((1, 6, 256), lambda s, e: (s, 0, 0)),
            pl.BlockSpec((1, 1, 256), lambda s, e: (s, 0, 0)),
            pl.BlockSpec((1, 256, 256), lambda s, e: (s, 0, 0)),
            pl.BlockSpec((1, 1, 256), lambda s, e: (s, 0, 0)),
        ],
        out_specs=pl.BlockSpec((1, BE, 256), lambda s, e: (s, e, 0)),
        out_shape=jax.ShapeDtypeStruct((2, EP, 256), jnp.float32),
    )(d6, W1s, b1s, W2s, b2s)


def _tc_d_body(hp0_ref, hp1_ref, dg0_ref, dg1_ref, wg_ref,
               y0_ref, y1_ref, dv0_ref, dv1_ref):
    h0 = hp0_ref[...].astype(jnp.float32)
    h1 = hp1_ref[...].astype(jnp.float32)
    wg = wg_ref[...]
    for s, (dg_ref, y_ref, dv_ref) in enumerate(
            ((dg0_ref, y0_ref, dv0_ref), (dg1_ref, y1_ref, dv1_ref))):
        deg = dg_ref[0] + dg_ref[1] + 1.0
        dinv = lax.rsqrt(deg)
        xw = (lax.dot_general(h0, wg[s, :256], (((1,), (0,)), ((), ())),
                              preferred_element_type=jnp.float32)
              + lax.dot_general(h1, wg[s, 256:], (((1,), (0,)), ((), ())),
                                preferred_element_type=jnp.float32))
        dvb = jnp.broadcast_to(dinv[:, None], xw.shape)
        y_ref[...] = dvb * xw
        dv_ref[...] = dvb


def _tc_d(hpad, degp, Wg):
    BN = 512
    G = NPAD // BN
    out = jax.ShapeDtypeStruct((NPAD, 128), jnp.float32)
    return pl.pallas_call(
        _tc_d_body,
        grid=(G,),
        in_specs=[
            pl.BlockSpec((BN, 256), lambda i: (i, 0)),
            pl.BlockSpec((BN, 256), lambda i: (i + G, 0)),
            pl.BlockSpec((2, BN), lambda i: (0, i)),
            pl.BlockSpec((2, BN), lambda i: (0, i + G)),
            pl.BlockSpec((2, 512, 128), lambda i: (0, 0, 0)),
        ],
        out_specs=[pl.BlockSpec((BN, 128), lambda i: (i, 0))] * 4,
        out_shape=[out, out, out, out],
    )(hpad, hpad, degp, degp, Wg)


def _tc_f_body(s0_ref, s1_ref, y0_ref, y1_ref, dv0_ref, dv1_ref,
               bg_ref, w3_ref, b3_ref, wct_ref, acc_ref):
    i = pl.program_id(0)
    g0 = jax.nn.relu(dv0_ref[...] * (s0_ref[0, 0] + s0_ref[1, 0]
                                     + y0_ref[...]) + bg_ref[0])
    g1 = jax.nn.relu(dv1_ref[...] * (s1_ref[0, 0] + s1_ref[1, 0]
                                     + y1_ref[...]) + bg_ref[1])
    h2 = jnp.concatenate([g0, g1], axis=1)
    h2 = jax.nn.relu(lax.dot_general(h2, w3_ref[...], (((1,), (0,)), ((), ())),
                                     preferred_element_type=jnp.float32)
                     + b3_ref[0])
    row = lax.broadcasted_iota(jnp.int32, (8, 128), 0)
    col = lax.broadcasted_iota(jnp.int32, (8, 128), 1)
    contrib = jnp.zeros((8, 128), jnp.float32)
    for c in range(3):
        val = jnp.sum(h2 * wct_ref[c])
        contrib = contrib + jnp.where((row == 0) & (col == c), val, 0.0)

    @pl.when(i == 0)
    def _():
        acc_ref[...] = contrib

    @pl.when(i > 0)
    def _():
        acc_ref[...] = acc_ref[...] + contrib


def _tc_f(S, y0, y1, dv0, dv1, bgs, W3, b3, WcT):
    BN = 512
    G = NPAD // BN
    return pl.pallas_call(
        _tc_f_body,
        grid=(G,),
        in_specs=[
            pl.BlockSpec((2, 1, BN, 128), lambda i: (0, 0, i, 0)),
            pl.BlockSpec((2, 1, BN, 128), lambda i: (0, 1, i, 0)),
            pl.BlockSpec((BN, 128), lambda i: (i, 0)),
            pl.BlockSpec((BN, 128), lambda i: (i, 0)),
            pl.BlockSpec((BN, 128), lambda i: (i, 0)),
            pl.BlockSpec((BN, 128), lambda i: (i, 0)),
            pl.BlockSpec((2, 128), lambda i: (0, 0)),
            pl.BlockSpec((256, 128), lambda i: (0, 0)),
            pl.BlockSpec((1, 128), lambda i: (0, 0)),
            pl.BlockSpec((3, BN, 128), lambda i: (0, i, 0)),
        ],
        out_specs=pl.BlockSpec((8, 128), lambda i: (0, 0)),
        out_shape=jax.ShapeDtypeStruct((8, 128), jnp.float32),
    )(S, S, y0, y1, dv0, dv1, bgs, W3, b3, WcT)


def kernel(pos_0, edge_index_0, batch_0, pos_1, edge_index_1, batch_1,
           W1_0, b1_0, W2_0, b2_0, W1_1, b1_1, W2_1, b2_1,
           Wg0, bg0, Wg1, bg1, W3, b3, Wc, bc):
    padN = EP - E
    srcs = jnp.stack([jnp.pad(edge_index_0[0], (0, padN)),
                      jnp.pad(edge_index_1[0], (0, padN))])
    dsts = jnp.stack([jnp.pad(edge_index_0[1], (0, padN), constant_values=N),
                      jnp.pad(edge_index_1[1], (0, padN), constant_values=N)])
    posT = jnp.stack([pos_0.T, pos_1.T])
    posT = jnp.pad(posT, ((0, 0), (0, 0), (0, POSP - N))).reshape(2, 3 * POSP)

    dstoff = (dsts + jnp.array([[0], [NPAD]], jnp.int32)).reshape(2, EP // 128, 128)
    dstflat = dsts.reshape(-1)
    srcr = (srcs + jnp.array([[0], [NPAD]], jnp.int32)).reshape(2, EP // 128, 128)
    dstr = dsts.reshape(2, EP // 128, 128)

    d6, degp = _sc_a(posT, srcs, dsts, dstoff)

    W1s = jnp.stack([W1_0, W1_1])
    b1s = jnp.stack([b1_0, b1_1]).reshape(2, 1, 256)
    W2s = jnp.stack([W2_0, W2_1])
    b2s = jnp.stack([b2_0, b2_1]).reshape(2, 1, 256)
    m = _tc_b(d6, W1s, b1s, W2s, b2s)
    mflat = m.reshape(2 * EP, 256)

    hpad = _sc_c(mflat, dstflat)

    Wg = jnp.stack([Wg0, Wg1])
    y0, y1, dv0, dv1 = _tc_d(hpad, degp, Wg)
    yflat = jnp.concatenate([y0, y1], axis=0)

    S = _sc_e(yflat, srcr, dstr)

    bgs = jnp.stack([bg0, bg1])
    WcT = (jnp.pad(Wc, ((0, (NPAD - N) * 128), (0, 0)))
           .reshape(NPAD, 128, 3).transpose(2, 0, 1))
    acc = _tc_f(S, y0, y1, dv0, dv1, bgs, W3, b3.reshape(1, 128), WcT)
    return acc[0, :3] + bc
